# Initial kernel scaffold; baseline (speedup 1.0000x reference)
import jax
import jax.numpy as jnp
from jax.experimental import pallas as pl


def kernel(xs_padded, lengths, ts, A, matrices, Q, E, Temb, W1, b1, W2):
    raise NotImplementedError("write your pallas kernel here")


# speedup vs baseline: 2.1869x; 2.1869x over previous
import jax
import jax.numpy as jnp
import numpy as np
from jax.experimental import pallas as pl
from jax.experimental.pallas import tpu as pltpu

_F32_TINY = float(np.finfo(np.float32).tiny)
_F32_TOP = float(1.0 - np.finfo(np.float32).eps)


def _body_a(ts_ref, len_ref, xs_ref, gt_ref, mt_ref, mtm1_ref, q_ref, e_ref,
            temb_ref, w1_ref, b1_ref, w2_ref,
            ce_ref, tu_ref, ex_ref, etxt_ref):
    k = pl.program_id(0)
    V = q_ref.shape[0]
    H = xs_ref.shape[2]

    l = len_ref[k]
    lf = l.astype(jnp.float32)

    xs = xs_ref[0]
    gt = gt_ref[0]
    mt = mt_ref[0]
    mtm1 = mtm1_ref[0]
    q = q_ref[...]
    e = e_ref[...]
    temb = temb_ref[0]
    w1 = w1_ref[...]
    b1 = b1_ref[...]
    w2 = w2_ref[...]

    iv = jax.lax.broadcasted_iota(jnp.int32, (V, H), 0)
    oht_xs = (iv == xs).astype(jnp.float32)

    xp = jax.lax.Precision.HIGHEST

    xdT = jnp.dot(mt, oht_xs, precision=xp)
    scores = jnp.log(jnp.clip(xdT, 1e-12)) + gt
    mx = jnp.max(scores, axis=0, keepdims=True)
    xt = jnp.min(jnp.where(scores == mx, iv, V), axis=0, keepdims=True)

    oht_xt = (iv == xt).astype(jnp.float32)
    etxt = jnp.dot(q, oht_xt, precision=xp).T
    gtm1 = jnp.dot(mtm1, oht_xs, precision=xp).T
    etxt_ref[0] = etxt
    tu_ref[0] = etxt * gtm1

    oh_xt = oht_xt.T
    hidden = jnp.dot(oh_xt, e, precision=xp) + temb
    a1 = jnp.maximum(jnp.dot(hidden, w1) + b1, 0.0)
    lg0 = jnp.dot(a1, w2)

    lgeps = lg0 + 1e-6
    rmax = jnp.max(lgeps, axis=1, keepdims=True)
    lse = rmax + jnp.log(jnp.sum(jnp.exp(lgeps - rmax), axis=1, keepdims=True))
    oh_xs = oht_xs.T
    take = jnp.sum(oh_xs * lgeps, axis=1, keepdims=True)
    ih_col = jax.lax.broadcasted_iota(jnp.int32, (H, 1), 0)
    mcol = (ih_col < l).astype(jnp.float32)
    ce_k = jnp.sum((lse - take) * mcol, keepdims=True).reshape(1, 1) / lf

    rmax0 = jnp.max(lg0, axis=1, keepdims=True)
    ex_ref[0] = jnp.exp(lg0 - rmax0)

    @pl.when(k == 0)
    def _():
        ce_ref[...] = jnp.zeros((1, 1), jnp.float32)

    ce_ref[...] += ce_k


def _body_b(ts_ref, len_ref, ex_ref, sex_ref, etxt_ref, mtm1_ref, a_ref,
            con_ref, pu_ref):
    k = pl.program_id(0)
    V = a_ref.shape[0]
    H = ex_ref.shape[1]

    l = len_ref[k]
    lf = l.astype(jnp.float32)

    ex = ex_ref[0]
    sex = sex_ref[0].T
    etxt = etxt_ref[0]
    mtm1 = mtm1_ref[0]
    a_mat = a_ref[...]

    p = ex / sex

    em1 = jax.lax.dot_general(p, mtm1, (((1,), (1,)), ((), ())))
    pu_ref[0] = etxt * em1

    lp = jnp.log(p + 1e-6)
    lga = jnp.dot(lp, a_mat)
    lga_next = pltpu.roll(lga, H - 1, 0)
    p_next = pltpu.roll(p, H - 1, 0)
    ih_col = jax.lax.broadcasted_iota(jnp.int32, (H, 1), 0)
    m2 = (ih_col < (l - 1)).astype(jnp.float32)
    d1 = jnp.sum(lga_next * p, axis=1, keepdims=True)
    d2 = jnp.sum(lga * p_next, axis=1, keepdims=True)
    con_k = -(jnp.sum((d1 + d2) * m2, keepdims=True).reshape(1, 1)
              ) / ((lf - 1.0) * V)

    @pl.when(k == 0)
    def _():
        con_ref[...] = jnp.zeros((1, 1), jnp.float32)

    con_ref[...] += con_k


def _body_c(len_ref, tu_ref, pu_ref, s_ref, sp_ref, term_ref):
    k = pl.program_id(0)
    H = tu_ref.shape[1]

    l = len_ref[k]

    tu = tu_ref[0]
    pu = pu_ref[0]
    s_col = s_ref[0].T
    sp_col = sp_ref[0].T

    tp = tu / s_col
    pp = pu / sp_col
    pl_ = jnp.log(jnp.clip(pp, _F32_TINY, _F32_TOP))

    safe_t = jnp.where(tp > 0, tp, 1.0)
    term = tp * jnp.log(safe_t) - tp * (pl_ + 1e-6)
    ih_col = jax.lax.broadcasted_iota(jnp.int32, (H, 1), 0)
    mcol = (ih_col < l).astype(jnp.float32)
    term_ref[0] = term * mcol


def kernel(xs_padded, lengths, ts, A, matrices, Q, E, Temb, W1, b1, W2):
    B_, H_ = xs_padded.shape
    V_ = A.shape[0]
    D_ = E.shape[1]
    DFF_ = W1.shape[1]

    g = jax.random.gumbel(jax.random.key(42), (B_ * H_, V_), jnp.float32)
    gT = g.reshape(B_, H_, V_).transpose(0, 2, 1)

    xs3 = xs_padded.astype(jnp.int32).reshape(B_, 1, H_)
    temb3 = Temb.reshape(Temb.shape[0], 1, D_)
    b1r = b1.reshape(1, DFF_)
    qstep = Q[1]
    ts32 = ts.astype(jnp.int32)
    len32 = lengths.astype(jnp.int32)

    big = jax.ShapeDtypeStruct((B_, H_, V_), jnp.float32)
    blk = lambda *_: (0, 0)

    grid_a = pltpu.PrefetchScalarGridSpec(
        num_scalar_prefetch=2,
        grid=(B_,),
        in_specs=[
            pl.BlockSpec((1, 1, H_), lambda k, ts_r, len_r: (k, 0, 0)),
            pl.BlockSpec((1, V_, H_), lambda k, ts_r, len_r: (k, 0, 0)),
            pl.BlockSpec((1, V_, V_), lambda k, ts_r, len_r: (ts_r[k], 0, 0)),
            pl.BlockSpec((1, V_, V_), lambda k, ts_r, len_r: (ts_r[k] - 1, 0, 0)),
            pl.BlockSpec((V_, V_), blk),
            pl.BlockSpec((V_, D_), blk),
            pl.BlockSpec((1, 1, D_), lambda k, ts_r, len_r: (ts_r[k], 0, 0)),
            pl.BlockSpec((D_, DFF_), blk),
            pl.BlockSpec((1, DFF_), blk),
            pl.BlockSpec((DFF_, V_), blk),
        ],
        out_specs=[
            pl.BlockSpec((1, 1), blk),
            pl.BlockSpec((1, H_, V_), lambda k, ts_r, len_r: (k, 0, 0)),
            pl.BlockSpec((1, H_, V_), lambda k, ts_r, len_r: (k, 0, 0)),
            pl.BlockSpec((1, H_, V_), lambda k, ts_r, len_r: (k, 0, 0)),
        ],
    )
    ce, tu3, ex3, etxt3 = pl.pallas_call(
        _body_a,
        grid_spec=grid_a,
        out_shape=[jax.ShapeDtypeStruct((1, 1), jnp.float32), big, big, big],
    )(ts32, len32, xs3, gT, matrices, matrices, qstep, E, temb3, W1, b1r, W2)

    sex3 = jnp.sum(ex3, axis=2).reshape(B_, 1, H_)
    s3 = jnp.sum(tu3, axis=2).reshape(B_, 1, H_)

    grid_b = pltpu.PrefetchScalarGridSpec(
        num_scalar_prefetch=2,
        grid=(B_,),
        in_specs=[
            pl.BlockSpec((1, H_, V_), lambda k, ts_r, len_r: (k, 0, 0)),
            pl.BlockSpec((1, 1, H_), lambda k, ts_r, len_r: (k, 0, 0)),
            pl.BlockSpec((1, H_, V_), lambda k, ts_r, len_r: (k, 0, 0)),
            pl.BlockSpec((1, V_, V_), lambda k, ts_r, len_r: (ts_r[k] - 1, 0, 0)),
            pl.BlockSpec((V_, V_), blk),
        ],
        out_specs=[
            pl.BlockSpec((1, 1), blk),
            pl.BlockSpec((1, H_, V_), lambda k, ts_r, len_r: (k, 0, 0)),
        ],
    )
    con, pu3 = pl.pallas_call(
        _body_b,
        grid_spec=grid_b,
        out_shape=[jax.ShapeDtypeStruct((1, 1), jnp.float32), big],
    )(ts32, len32, ex3, sex3, etxt3, matrices, A)

    sp3 = jnp.sum(pu3, axis=2).reshape(B_, 1, H_)

    grid_c = pltpu.PrefetchScalarGridSpec(
        num_scalar_prefetch=1,
        grid=(B_,),
        in_specs=[
            pl.BlockSpec((1, H_, V_), lambda k, len_r: (k, 0, 0)),
            pl.BlockSpec((1, H_, V_), lambda k, len_r: (k, 0, 0)),
            pl.BlockSpec((1, 1, H_), lambda k, len_r: (k, 0, 0)),
            pl.BlockSpec((1, 1, H_), lambda k, len_r: (k, 0, 0)),
        ],
        out_specs=[pl.BlockSpec((1, H_, V_), lambda k, len_r: (k, 0, 0))],
    )
    (term3,) = pl.pallas_call(
        _body_c,
        grid_spec=grid_c,
        out_shape=[big],
    )(len32, tu3, pu3, s3, sp3)

    kl = 0.0
    for k in range(B_):
        kl = kl + jnp.sum(term3[k]) / lengths[k].astype(jnp.float32)

    return (kl, ce[0, 0], con[0, 0] / B_ * 100.0)
